# baseline (speedup 1.0000x reference)
import functools

import jax
import jax.numpy as jnp
from jax import lax
from jax.experimental import pallas as pl
from jax.experimental.pallas import tpu as pltpu
from jax.experimental.pallas import tpu_sc as plsc

N = 10000
E = 320000
D = 128

NC = 2
NS = 16
NW = NC * NS
CH = 128
NJ = 80
NJB = 80
NBLK = NJ // NJB
NB = 1
EP = NW * NJ * CH
NACC = N + 8
TRASH = N
PER = 624
TAIL = N - NS * PER


def _sc_aggregate(x, row3, col3):
    mesh = plsc.VectorSubcoreMesh(
        core_axis_name="c", subcore_axis_name="s",
        num_cores=NC, num_subcores=NS,
    )

    @functools.partial(
        pl.kernel,
        mesh=mesh,
        out_type=jax.ShapeDtypeStruct((NC, N, D), jnp.float32),
        scratch_types=[
            pltpu.VMEM((NJB, CH), jnp.int32),
            pltpu.VMEM((NJB, CH), jnp.int32),
            *[pltpu.VMEM((CH, D), jnp.float32) for _ in range(NB)],
            pltpu.VMEM_SHARED((NACC, D), jnp.float32),
            pltpu.SemaphoreType.DMA,
            *[pltpu.SemaphoreType.DMA for _ in range(NB)],
            *[pltpu.SemaphoreType.DMA for _ in range(NB)],
        ],
    )
    def sc_kernel(x_hbm, row_hbm, col_hbm, out_hbm, dstb, colb,
                  *rest):
        gath = rest[:NB]
        acc_sh = rest[NB]
        sem_seed = rest[NB + 1]
        sg = rest[NB + 2:NB + 2 + NB]
        ss = rest[NB + 2 + NB:NB + 2 + 2 * NB]
        core = lax.axis_index("c")
        sid = lax.axis_index("s")
        wid = sid * NC + core
        base = sid * PER

        pltpu.async_copy(x_hbm.at[pl.ds(base, PER)],
                         acc_sh.at[pl.ds(base, PER)], sem_seed)

        @pl.when(sid == 0)
        def _():
            pltpu.async_copy(x_hbm.at[pl.ds(NS * PER, TAIL)],
                             acc_sh.at[pl.ds(NS * PER, TAIL)], sem_seed)

        pltpu.make_async_copy(x_hbm.at[pl.ds(base, PER)],
                              acc_sh.at[pl.ds(base, PER)], sem_seed).wait()

        @pl.when(sid == 0)
        def _():
            pltpu.make_async_copy(x_hbm.at[pl.ds(NS * PER, TAIL)],
                                  acc_sh.at[pl.ds(NS * PER, TAIL)],
                                  sem_seed).wait()

        plsc.subcore_barrier()

        def g_start(j, b):
            pltpu.async_copy(x_hbm.at[colb.at[j]], gath[b], sg[b])

        def g_wait(b):
            pltpu.make_async_copy(x_hbm.at[colb.at[0]], gath[b],
                                  sg[b]).wait()

        def s_start(j, b):
            pltpu.async_copy(gath[b], acc_sh.at[dstb.at[j]], ss[b],
                             add=True)

        def s_wait(b):
            pltpu.make_async_copy(gath[b], acc_sh.at[dstb.at[0]],
                                  ss[b]).wait()

        @pl.loop(0, NBLK)
        def _(blk):
            pltpu.sync_copy(row_hbm.at[wid, pl.ds(blk * NJB, NJB)], dstb)
            pltpu.sync_copy(col_hbm.at[wid, pl.ds(blk * NJB, NJB)], colb)

            @pl.loop(0, NJB)
            def _(j):
                for i in range(CH // 16):
                    r = dstb[j, pl.ds(i * 16, 16)]
                    c = colb[j, pl.ds(i * 16, 16)]
                    dstb[j, pl.ds(i * 16, 16)] = jnp.where(r == c, TRASH, r)

            @pl.loop(0, NJB)
            def _(j):
                g_start(j, 0)
                g_wait(0)
                s_start(j, 0)
                s_wait(0)

        plsc.subcore_barrier()

        pltpu.sync_copy(acc_sh.at[pl.ds(base, PER)],
                        out_hbm.at[core, pl.ds(base, PER)])

        @pl.when(sid == 0)
        def _():
            pltpu.sync_copy(acc_sh.at[pl.ds(NS * PER, TAIL)],
                            out_hbm.at[core, pl.ds(NS * PER, TAIL)])

    return sc_kernel(x, row3, col3)


def _tc_finish(p0, p1, x, W):
    BR = 1000

    def body(p0_ref, p1_ref, x_ref, w_ref, o_ref):
        s = p0_ref[...] + p1_ref[...] - x_ref[...]
        y = jnp.dot(s, w_ref[...], preferred_element_type=jnp.float32)
        nrm = jnp.sqrt(jnp.sum(y * y, axis=1, keepdims=True))
        o_ref[...] = y / jnp.maximum(nrm, 1e-12)

    row_spec = pl.BlockSpec((BR, D), lambda i: (i, 0))
    return pl.pallas_call(
        body,
        grid=(N // BR,),
        in_specs=[row_spec, row_spec, row_spec,
                  pl.BlockSpec((D, D), lambda i: (0, 0))],
        out_specs=row_spec,
        out_shape=jax.ShapeDtypeStruct((N, D), jnp.float32),
    )(p0, p1, x, W)


@jax.jit
def kernel(x, edge_index, W):
    pad = EP - E
    padz = jnp.zeros((pad,), jnp.int32)
    row3 = jnp.concatenate([edge_index[0], padz]).reshape(NW, NJ, CH)
    col3 = jnp.concatenate([edge_index[1], padz]).reshape(NW, NJ, CH)
    partials = _sc_aggregate(x, row3, col3)
    return _tc_finish(partials[0], partials[1], x, W)


# speedup vs baseline: 1.0828x; 1.0828x over previous
import functools

import jax
import jax.numpy as jnp
from jax import lax
from jax.experimental import pallas as pl
from jax.experimental.pallas import tpu as pltpu
from jax.experimental.pallas import tpu_sc as plsc

N = 10000
E = 320000
D = 128

NC = 2
NS = 16
NW = NC * NS
CH = 128
NJ = 80
NJB = 40
NBLK = NJ // NJB
NB = 2
EP = NW * NJ * CH
NACC = N + 8
TRASH = N
PER = 624
TAIL = N - NS * PER


def _sc_aggregate(x, row3, col3):
    mesh = plsc.VectorSubcoreMesh(
        core_axis_name="c", subcore_axis_name="s",
        num_cores=NC, num_subcores=NS,
    )

    @functools.partial(
        pl.kernel,
        mesh=mesh,
        out_type=jax.ShapeDtypeStruct((NC, N, D), jnp.float32),
        scratch_types=[
            pltpu.VMEM((NJB, CH), jnp.int32),
            pltpu.VMEM((NJB, CH), jnp.int32),
            *[pltpu.VMEM((CH, D), jnp.float32) for _ in range(NB)],
            pltpu.VMEM_SHARED((NACC, D), jnp.float32),
            pltpu.SemaphoreType.DMA,
            *[pltpu.SemaphoreType.DMA for _ in range(NB)],
            *[pltpu.SemaphoreType.DMA for _ in range(NB)],
        ],
    )
    def sc_kernel(x_hbm, row_hbm, col_hbm, out_hbm, dstb, colb,
                  *rest):
        gath = rest[:NB]
        acc_sh = rest[NB]
        sem_seed = rest[NB + 1]
        sg = rest[NB + 2:NB + 2 + NB]
        ss = rest[NB + 2 + NB:NB + 2 + 2 * NB]
        core = lax.axis_index("c")
        sid = lax.axis_index("s")
        wid = sid * NC + core
        base = sid * PER

        pltpu.async_copy(x_hbm.at[pl.ds(base, PER)],
                         acc_sh.at[pl.ds(base, PER)], sem_seed)

        @pl.when(sid == 0)
        def _():
            pltpu.async_copy(x_hbm.at[pl.ds(NS * PER, TAIL)],
                             acc_sh.at[pl.ds(NS * PER, TAIL)], sem_seed)

        pltpu.make_async_copy(x_hbm.at[pl.ds(base, PER)],
                              acc_sh.at[pl.ds(base, PER)], sem_seed).wait()

        @pl.when(sid == 0)
        def _():
            pltpu.make_async_copy(x_hbm.at[pl.ds(NS * PER, TAIL)],
                                  acc_sh.at[pl.ds(NS * PER, TAIL)],
                                  sem_seed).wait()

        plsc.subcore_barrier()

        def g_start(j, b):
            pltpu.async_copy(x_hbm.at[colb.at[j]], gath[b], sg[b])

        def g_wait(b):
            pltpu.make_async_copy(x_hbm.at[colb.at[0]], gath[b],
                                  sg[b]).wait()

        def s_start(j, b):
            pltpu.async_copy(gath[b], acc_sh.at[dstb.at[j]], ss[b],
                             add=True)

        def s_wait(b):
            pltpu.make_async_copy(gath[b], acc_sh.at[dstb.at[0]],
                                  ss[b]).wait()

        @pl.loop(0, NBLK)
        def _(blk):
            pltpu.sync_copy(row_hbm.at[wid, pl.ds(blk * NJB, NJB)], dstb)
            pltpu.sync_copy(col_hbm.at[wid, pl.ds(blk * NJB, NJB)], colb)

            for j in range(NB - 1):
                g_start(j, j)
            for j in range(NJB):
                b = j % NB
                g_wait(b)
                s_start(j, b)
                nxt = j + NB - 1
                if nxt < NJB:
                    nb = nxt % NB
                    if nxt >= NB:
                        s_wait(nb)
                    g_start(nxt, nb)
            for j in range(NJB - NB + 1, NJB + 1):
                s_wait(j % NB)

        plsc.subcore_barrier()

        pltpu.sync_copy(acc_sh.at[pl.ds(base, PER)],
                        out_hbm.at[core, pl.ds(base, PER)])

        @pl.when(sid == 0)
        def _():
            pltpu.sync_copy(acc_sh.at[pl.ds(NS * PER, TAIL)],
                            out_hbm.at[core, pl.ds(NS * PER, TAIL)])

    return sc_kernel(x, row3, col3)


def _tc_finish(p0, p1, x, W):
    BR = 1000

    def body(p0_ref, p1_ref, x_ref, w_ref, o_ref):
        s = p0_ref[...] + p1_ref[...] - x_ref[...]
        y = jnp.dot(s, w_ref[...], preferred_element_type=jnp.float32)
        nrm = jnp.sqrt(jnp.sum(y * y, axis=1, keepdims=True))
        o_ref[...] = y / jnp.maximum(nrm, 1e-12)

    row_spec = pl.BlockSpec((BR, D), lambda i: (i, 0))
    return pl.pallas_call(
        body,
        grid=(N // BR,),
        in_specs=[row_spec, row_spec, row_spec,
                  pl.BlockSpec((D, D), lambda i: (0, 0))],
        out_specs=row_spec,
        out_shape=jax.ShapeDtypeStruct((N, D), jnp.float32),
    )(p0, p1, x, W)


@jax.jit
def kernel(x, edge_index, W):
    pad = EP - E
    padz = jnp.zeros((pad,), jnp.int32)
    row = jnp.concatenate([edge_index[0], padz])
    col = jnp.concatenate([edge_index[1], padz])
    dst = jnp.where(row == col, TRASH, row)
    row3 = dst.reshape(NW, NJ, CH)
    col3 = col.reshape(NW, NJ, CH)
    partials = _sc_aggregate(x, row3, col3)
    return _tc_finish(partials[0], partials[1], x, W)
